# baseline (speedup 1.0000x reference)
import functools

import jax
import jax.numpy as jnp
from jax import lax
from jax.experimental import pallas as pl
from jax.experimental.pallas import tpu as pltpu
from jax.experimental.pallas import tpu_sc as plsc

N = 10000
E = 320000
CHUNK = 112
NW = 32
NC0 = 75
NC1 = 104
NCHUNK = NC1
E_PAD = 16 * (NC0 + NC1) * CHUNK
N_PAD = 10240
ROWS_PER_TILE = N_PAD // 16

_GATHER_DNUMS = lax.GatherDimensionNumbers(
    offset_dims=(), collapsed_slice_dims=(0,), start_index_map=(0,))


def _lane_splat(vec, l):
    idx = jnp.broadcast_to(l, (16, 1)).astype(jnp.int32)
    return lax.gather(vec, idx, _GATHER_DNUMS, slice_sizes=(1,),
                      mode=lax.GatherScatterMode.PROMISE_IN_BOUNDS)


def _make_agg(D):
    ngrp = D // 16
    mesh = plsc.VectorSubcoreMesh(core_axis_name="c", subcore_axis_name="s")

    @functools.partial(
        pl.kernel,
        out_type=jax.ShapeDtypeStruct((2, N_PAD, D), jnp.float32),
        mesh=mesh,
        scratch_types=[
            pltpu.VMEM((4, 2, CHUNK), jnp.int32),
            pltpu.VMEM((4, CHUNK), jnp.float32),
            pltpu.VMEM((3, CHUNK, D), jnp.float32),
            pltpu.VMEM_SHARED((N_PAD, D), jnp.float32),
            pltpu.SemaphoreType.DMA,
            pltpu.SemaphoreType.DMA,
        ],
    )
    def agg(m_hbm, sd_hbm, ew_hbm, out_hbm, sd_v, ew_v, rows_v, acc,
            isem, gsem):
        c = lax.axis_index("c")
        s = lax.axis_index("s")
        wid = c * 16 + s

        zero = jnp.zeros((16,), jnp.float32)

        def zrow(i, carry):
            for j in range(ngrp):
                rows_v[0, i, pl.ds(j * 16, 16)] = zero
            return carry

        lax.fori_loop(0, CHUNK, zrow, 0)
        r0 = s * ROWS_PER_TILE
        for k in range(5):
            pltpu.sync_copy(rows_v.at[0], acc.at[pl.ds(r0 + k * CHUNK, CHUNK)])
        pltpu.sync_copy(rows_v.at[0, pl.ds(0, 80)],
                        acc.at[pl.ds(r0 + 5 * CHUNK, 80)])
        plsc.subcore_barrier()

        def idxload(jc):
            return (pltpu.make_async_copy(
                        sd_hbm.at[wid, jc], sd_v.at[lax.rem(jc, 4)], isem),
                    pltpu.make_async_copy(
                        ew_hbm.at[wid, jc], ew_v.at[lax.rem(jc, 4)], isem))

        def gather(jc, b):
            return pltpu.make_async_copy(
                m_hbm.at[sd_v.at[lax.rem(jc, 4), 0]], rows_v.at[b], gsem)

        def istart(jc):
            a, bb = idxload(jc)
            a.start()
            bb.start()

        def iwait(jc):
            a, bb = idxload(jc)
            a.wait()
            bb.wait()

        jlim = jnp.where(c == 0, NC0, NC1)

        istart(0)
        iwait(0)
        gather(0, 0).start()
        istart(1)
        iwait(1)
        gather(1, 1).start()
        istart(2)

        def body(jc, carry):
            b = lax.rem(jc, 3)
            r = lax.rem(jc, 4)

            gather(jc, b).wait()

            @pl.when(jc + 2 < jlim)
            def _():
                iwait(jc + 2)
                gather(jc + 2, lax.rem(jc + 2, 3)).start()

                @pl.when(jc + 3 < jlim)
                def _():
                    istart(jc + 3)

            @plsc.parallel_loop(0, CHUNK, step=1, unroll=4)
            def scale_edge(e):
                l = jnp.bitwise_and(e, 15)
                ew_vec = ew_v[r, pl.ds(e - l, 16)]
                sc = _lane_splat(ew_vec, l)
                for j in range(ngrp):
                    rows_v[b, e, pl.ds(j * 16, 16)] = (
                        rows_v[b, e, pl.ds(j * 16, 16)] * sc)

            pltpu.sync_copy(rows_v.at[b], acc.at[sd_v.at[r, 1]], add=True)
            return carry

        lax.fori_loop(0, jlim, body, 0)
        plsc.subcore_barrier()

        for k in range(5):
            pltpu.sync_copy(acc.at[pl.ds(r0 + k * CHUNK, CHUNK)], rows_v.at[0])
            pltpu.sync_copy(rows_v.at[0],
                            out_hbm.at[c, pl.ds(r0 + k * CHUNK, CHUNK)])
        pltpu.sync_copy(acc.at[pl.ds(r0 + 5 * CHUNK, 80)],
                        rows_v.at[0, pl.ds(0, 80)])
        pltpu.sync_copy(rows_v.at[0, pl.ds(0, 80)],
                        out_hbm.at[c, pl.ds(r0 + 5 * CHUNK, 80)])

    return agg


_agg128 = _make_agg(128)

_RB = 2000


def _mm_body(x_ref, w_ref, o_ref):
    o_ref[...] = jnp.dot(x_ref[...], w_ref[...],
                         preferred_element_type=jnp.float32)


def _matmul(x, w):
    n, d = x.shape
    do = w.shape[1]
    return pl.pallas_call(
        _mm_body,
        grid=(n // _RB,),
        in_specs=[
            pl.BlockSpec((_RB, d), lambda i: (i, 0)),
            pl.BlockSpec((d, do), lambda i: (0, 0)),
        ],
        out_specs=pl.BlockSpec((_RB, do), lambda i: (i, 0)),
        out_shape=jax.ShapeDtypeStruct((n, do), jnp.float32),
    )(x, w)


def _fused_body(p_ref, b_ref, w_ref, o_ref):
    h = jax.nn.relu(p_ref[0] + p_ref[1] + b_ref[...])
    o_ref[...] = jnp.dot(h, w_ref[...], preferred_element_type=jnp.float32)


def _fused(p, b, w):
    d = p.shape[2]
    do = w.shape[1]
    return pl.pallas_call(
        _fused_body,
        grid=(N // _RB,),
        in_specs=[
            pl.BlockSpec((2, _RB, d), lambda i: (0, i, 0)),
            pl.BlockSpec((1, d), lambda i: (0, 0)),
            pl.BlockSpec((d, do), lambda i: (0, 0)),
        ],
        out_specs=pl.BlockSpec((_RB, do), lambda i: (i, 0)),
        out_shape=jax.ShapeDtypeStruct((N, do), jnp.float32),
    )(p, b.reshape(1, d), w)


def _act_body(p_ref, b_ref, o_ref):
    o_ref[...] = jax.nn.relu(p_ref[0] + p_ref[1] + b_ref[...])


def _act(p, b):
    d = p.shape[2]
    return pl.pallas_call(
        _act_body,
        grid=(N // _RB,),
        in_specs=[
            pl.BlockSpec((2, _RB, d), lambda i: (0, i, 0)),
            pl.BlockSpec((1, d), lambda i: (0, 0)),
        ],
        out_specs=pl.BlockSpec((_RB, d), lambda i: (i, 0)),
        out_shape=jax.ShapeDtypeStruct((N, d), jnp.float32),
    )(p, b.reshape(1, d))


def _mm_final_body(p_ref, w_ref, b_ref, o_ref):
    o_ref[...] = jnp.dot(p_ref[0] + p_ref[1], w_ref[...],
                         preferred_element_type=jnp.float32) + b_ref[...]


def _mm_final(p, w, b):
    d = p.shape[2]
    do = w.shape[1]
    return pl.pallas_call(
        _mm_final_body,
        grid=(N // _RB,),
        in_specs=[
            pl.BlockSpec((2, _RB, d), lambda i: (0, i, 0)),
            pl.BlockSpec((d, do), lambda i: (0, 0)),
            pl.BlockSpec((1, do), lambda i: (0, 0)),
        ],
        out_specs=pl.BlockSpec((_RB, do), lambda i: (i, 0)),
        out_shape=jax.ShapeDtypeStruct((N, do), jnp.float32),
    )(p, w, b.reshape(1, do))


def kernel(features, edge_index, edge_weight, W0, b0, W1, b1, W2, b2):
    src = edge_index[0].astype(jnp.int32)
    dst = edge_index[1].astype(jnp.int32)
    pad = E_PAD - E
    src = jnp.concatenate([src, jnp.zeros((pad,), jnp.int32)])
    dst = jnp.concatenate([dst, jnp.zeros((pad,), jnp.int32)])
    ew = jnp.concatenate([edge_weight.astype(jnp.float32),
                          jnp.zeros((pad,), jnp.float32)])

    def split(x):
        e0 = 16 * NC0 * CHUNK
        x0 = x[:e0].reshape(16, NC0, CHUNK)
        x0 = jnp.pad(x0, ((0, 0), (0, NCHUNK - NC0), (0, 0)))
        x1 = x[e0:].reshape(16, NC1, CHUNK)
        return jnp.concatenate([x0, x1], axis=0)

    sd = jnp.stack([split(src), split(dst)], axis=2)
    ew = split(ew)

    m0 = _matmul(features, W0)
    p0 = _agg128(m0, sd, ew)
    m1 = _fused(p0, b0, W1)
    p1 = _agg128(m1, sd, ew)
    h2 = _act(p1, b1)
    p2 = _agg128(h2, sd, ew)
    return _mm_final(p2, W2, b2)


# speedup vs baseline: 1.0929x; 1.0929x over previous
import functools

import jax
import jax.numpy as jnp
from jax import lax
from jax.experimental import pallas as pl
from jax.experimental.pallas import tpu as pltpu
from jax.experimental.pallas import tpu_sc as plsc

N = 10000
E = 320000
CHUNK = 112
NW = 32
NC0 = 80
NC1 = 99
NCHUNK = NC1
E_PAD = 16 * (NC0 + NC1) * CHUNK
N_PAD = 10240
ROWS_PER_TILE = N_PAD // 16

_GATHER_DNUMS = lax.GatherDimensionNumbers(
    offset_dims=(), collapsed_slice_dims=(0,), start_index_map=(0,))


def _lane_splat(vec, l):
    idx = jnp.broadcast_to(l, (16, 1)).astype(jnp.int32)
    return lax.gather(vec, idx, _GATHER_DNUMS, slice_sizes=(1,),
                      mode=lax.GatherScatterMode.PROMISE_IN_BOUNDS)


def _make_agg(D):
    ngrp = D // 16
    mesh = plsc.VectorSubcoreMesh(core_axis_name="c", subcore_axis_name="s")

    @functools.partial(
        pl.kernel,
        out_type=jax.ShapeDtypeStruct((2, N_PAD, D), jnp.float32),
        mesh=mesh,
        scratch_types=[
            pltpu.VMEM((4, 2, CHUNK), jnp.int32),
            pltpu.VMEM((4, CHUNK), jnp.float32),
            pltpu.VMEM((3, CHUNK, D), jnp.float32),
            pltpu.VMEM_SHARED((N_PAD, D), jnp.float32),
            pltpu.SemaphoreType.DMA,
            pltpu.SemaphoreType.DMA,
        ],
    )
    def agg(m_hbm, sd_hbm, ew_hbm, out_hbm, sd_v, ew_v, rows_v, acc,
            isem, gsem):
        c = lax.axis_index("c")
        s = lax.axis_index("s")
        wid = c * 16 + s

        zero = jnp.zeros((16,), jnp.float32)

        def zrow(i, carry):
            for j in range(ngrp):
                rows_v[0, i, pl.ds(j * 16, 16)] = zero
            return carry

        lax.fori_loop(0, CHUNK, zrow, 0)
        r0 = s * ROWS_PER_TILE
        for k in range(5):
            pltpu.sync_copy(rows_v.at[0], acc.at[pl.ds(r0 + k * CHUNK, CHUNK)])
        pltpu.sync_copy(rows_v.at[0, pl.ds(0, 80)],
                        acc.at[pl.ds(r0 + 5 * CHUNK, 80)])
        plsc.subcore_barrier()

        def idxload(jc):
            return (pltpu.make_async_copy(
                        sd_hbm.at[wid, jc], sd_v.at[lax.rem(jc, 4)], isem),
                    pltpu.make_async_copy(
                        ew_hbm.at[wid, jc], ew_v.at[lax.rem(jc, 4)], isem))

        def gather(jc, b):
            return pltpu.make_async_copy(
                m_hbm.at[sd_v.at[lax.rem(jc, 4), 0]], rows_v.at[b], gsem)

        def istart(jc):
            a, bb = idxload(jc)
            a.start()
            bb.start()

        def iwait(jc):
            a, bb = idxload(jc)
            a.wait()
            bb.wait()

        jlim = jnp.where(c == 0, NC0, NC1)

        istart(0)
        iwait(0)
        gather(0, 0).start()
        istart(1)
        iwait(1)
        gather(1, 1).start()
        istart(2)

        def body(jc, carry):
            b = lax.rem(jc, 3)
            r = lax.rem(jc, 4)

            gather(jc, b).wait()

            @pl.when(jc + 2 < jlim)
            def _():
                iwait(jc + 2)
                gather(jc + 2, lax.rem(jc + 2, 3)).start()

                @pl.when(jc + 3 < jlim)
                def _():
                    istart(jc + 3)

            @plsc.parallel_loop(0, CHUNK, step=1, unroll=4)
            def scale_edge(e):
                l = jnp.bitwise_and(e, 15)
                ew_vec = ew_v[r, pl.ds(e - l, 16)]
                sc = _lane_splat(ew_vec, l)
                for j in range(ngrp):
                    rows_v[b, e, pl.ds(j * 16, 16)] = (
                        rows_v[b, e, pl.ds(j * 16, 16)] * sc)

            pltpu.sync_copy(rows_v.at[b], acc.at[sd_v.at[r, 1]], add=True)
            return carry

        lax.fori_loop(0, jlim, body, 0)
        plsc.subcore_barrier()

        for k in range(5):
            pltpu.sync_copy(acc.at[pl.ds(r0 + k * CHUNK, CHUNK)], rows_v.at[0])
            pltpu.sync_copy(rows_v.at[0],
                            out_hbm.at[c, pl.ds(r0 + k * CHUNK, CHUNK)])
        pltpu.sync_copy(acc.at[pl.ds(r0 + 5 * CHUNK, 80)],
                        rows_v.at[0, pl.ds(0, 80)])
        pltpu.sync_copy(rows_v.at[0, pl.ds(0, 80)],
                        out_hbm.at[c, pl.ds(r0 + 5 * CHUNK, 80)])

    return agg


_agg128 = _make_agg(128)

_RB = 2000


def _mm_body(x_ref, w_ref, o_ref):
    o_ref[...] = jnp.dot(x_ref[...], w_ref[...],
                         preferred_element_type=jnp.float32)


def _matmul(x, w):
    n, d = x.shape
    do = w.shape[1]
    return pl.pallas_call(
        _mm_body,
        grid=(n // _RB,),
        in_specs=[
            pl.BlockSpec((_RB, d), lambda i: (i, 0)),
            pl.BlockSpec((d, do), lambda i: (0, 0)),
        ],
        out_specs=pl.BlockSpec((_RB, do), lambda i: (i, 0)),
        out_shape=jax.ShapeDtypeStruct((n, do), jnp.float32),
    )(x, w)


def _fused_body(p_ref, b_ref, w_ref, o_ref):
    h = jax.nn.relu(p_ref[0] + p_ref[1] + b_ref[...])
    o_ref[...] = jnp.dot(h, w_ref[...], preferred_element_type=jnp.float32)


def _fused(p, b, w):
    d = p.shape[2]
    do = w.shape[1]
    return pl.pallas_call(
        _fused_body,
        grid=(N // _RB,),
        in_specs=[
            pl.BlockSpec((2, _RB, d), lambda i: (0, i, 0)),
            pl.BlockSpec((1, d), lambda i: (0, 0)),
            pl.BlockSpec((d, do), lambda i: (0, 0)),
        ],
        out_specs=pl.BlockSpec((_RB, do), lambda i: (i, 0)),
        out_shape=jax.ShapeDtypeStruct((N, do), jnp.float32),
    )(p, b.reshape(1, d), w)


def _act_body(p_ref, b_ref, o_ref):
    o_ref[...] = jax.nn.relu(p_ref[0] + p_ref[1] + b_ref[...])


def _act(p, b):
    d = p.shape[2]
    return pl.pallas_call(
        _act_body,
        grid=(N // _RB,),
        in_specs=[
            pl.BlockSpec((2, _RB, d), lambda i: (0, i, 0)),
            pl.BlockSpec((1, d), lambda i: (0, 0)),
        ],
        out_specs=pl.BlockSpec((_RB, d), lambda i: (i, 0)),
        out_shape=jax.ShapeDtypeStruct((N, d), jnp.float32),
    )(p, b.reshape(1, d))


def _mm_final_body(p_ref, w_ref, b_ref, o_ref):
    o_ref[...] = jnp.dot(p_ref[0] + p_ref[1], w_ref[...],
                         preferred_element_type=jnp.float32) + b_ref[...]


def _mm_final(p, w, b):
    d = p.shape[2]
    do = w.shape[1]
    return pl.pallas_call(
        _mm_final_body,
        grid=(N // _RB,),
        in_specs=[
            pl.BlockSpec((2, _RB, d), lambda i: (0, i, 0)),
            pl.BlockSpec((d, do), lambda i: (0, 0)),
            pl.BlockSpec((1, do), lambda i: (0, 0)),
        ],
        out_specs=pl.BlockSpec((_RB, do), lambda i: (i, 0)),
        out_shape=jax.ShapeDtypeStruct((N, do), jnp.float32),
    )(p, w, b.reshape(1, do))


def kernel(features, edge_index, edge_weight, W0, b0, W1, b1, W2, b2):
    src = edge_index[0].astype(jnp.int32)
    dst = edge_index[1].astype(jnp.int32)
    pad = E_PAD - E
    src = jnp.concatenate([src, jnp.zeros((pad,), jnp.int32)])
    dst = jnp.concatenate([dst, jnp.zeros((pad,), jnp.int32)])
    ew = jnp.concatenate([edge_weight.astype(jnp.float32),
                          jnp.zeros((pad,), jnp.float32)])

    def split(x):
        e0 = 16 * NC0 * CHUNK
        x0 = x[:e0].reshape(16, NC0, CHUNK)
        x0 = jnp.pad(x0, ((0, 0), (0, NCHUNK - NC0), (0, 0)))
        x1 = x[e0:].reshape(16, NC1, CHUNK)
        return jnp.concatenate([x0, x1], axis=0)

    sd = jnp.stack([split(src), split(dst)], axis=2)
    ew = split(ew)

    m0 = _matmul(features, W0)
    p0 = _agg128(m0, sd, ew)
    m1 = _fused(p0, b0, W1)
    p1 = _agg128(m1, sd, ew)
    h2 = _act(p1, b1)
    p2 = _agg128(h2, sd, ew)
    return _mm_final(p2, W2, b2)


# speedup vs baseline: 1.1152x; 1.0204x over previous
import functools

import jax
import jax.numpy as jnp
from jax import lax
from jax.experimental import pallas as pl
from jax.experimental.pallas import tpu as pltpu
from jax.experimental.pallas import tpu_sc as plsc

N = 10000
E = 320000
CHUNK = 112
NW = 32
NC0 = 78
NC1 = 101
NCHUNK = NC1
E_PAD = 16 * (NC0 + NC1) * CHUNK
N_PAD = 10240
ROWS_PER_TILE = N_PAD // 16

_GATHER_DNUMS = lax.GatherDimensionNumbers(
    offset_dims=(), collapsed_slice_dims=(0,), start_index_map=(0,))


def _lane_splat(vec, l):
    idx = jnp.broadcast_to(l, (16, 1)).astype(jnp.int32)
    return lax.gather(vec, idx, _GATHER_DNUMS, slice_sizes=(1,),
                      mode=lax.GatherScatterMode.PROMISE_IN_BOUNDS)


def _make_agg(D):
    ngrp = D // 16
    mesh = plsc.VectorSubcoreMesh(core_axis_name="c", subcore_axis_name="s")

    @functools.partial(
        pl.kernel,
        out_type=jax.ShapeDtypeStruct((2, N_PAD, D), jnp.float32),
        mesh=mesh,
        scratch_types=[
            pltpu.VMEM((4, 2, CHUNK), jnp.int32),
            pltpu.VMEM((4, CHUNK), jnp.float32),
            pltpu.VMEM((3, CHUNK, D), jnp.float32),
            pltpu.VMEM_SHARED((N_PAD, D), jnp.float32),
            pltpu.SemaphoreType.DMA,
            pltpu.SemaphoreType.DMA,
        ],
    )
    def agg(m_hbm, sd_hbm, ew_hbm, out_hbm, sd_v, ew_v, rows_v, acc,
            isem, gsem):
        c = lax.axis_index("c")
        s = lax.axis_index("s")
        wid = c * 16 + s

        zero = jnp.zeros((16,), jnp.float32)

        def zrow(i, carry):
            for j in range(ngrp):
                rows_v[0, i, pl.ds(j * 16, 16)] = zero
            return carry

        lax.fori_loop(0, CHUNK, zrow, 0)
        r0 = s * ROWS_PER_TILE
        for k in range(5):
            pltpu.sync_copy(rows_v.at[0], acc.at[pl.ds(r0 + k * CHUNK, CHUNK)])
        pltpu.sync_copy(rows_v.at[0, pl.ds(0, 80)],
                        acc.at[pl.ds(r0 + 5 * CHUNK, 80)])
        plsc.subcore_barrier()

        def idxload(jc):
            return (pltpu.make_async_copy(
                        sd_hbm.at[wid, jc], sd_v.at[lax.rem(jc, 4)], isem),
                    pltpu.make_async_copy(
                        ew_hbm.at[wid, jc], ew_v.at[lax.rem(jc, 4)], isem))

        def gather(jc, b):
            return pltpu.make_async_copy(
                m_hbm.at[sd_v.at[lax.rem(jc, 4), 0]], rows_v.at[b], gsem)

        def istart(jc):
            a, bb = idxload(jc)
            a.start()
            bb.start()

        def iwait(jc):
            a, bb = idxload(jc)
            a.wait()
            bb.wait()

        jlim = jnp.where(c == 0, NC0, NC1)

        istart(0)
        iwait(0)
        gather(0, 0).start()
        istart(1)
        iwait(1)
        gather(1, 1).start()
        istart(2)

        def body(jc, carry):
            b = lax.rem(jc, 3)
            r = lax.rem(jc, 4)

            gather(jc, b).wait()

            @pl.when(jc + 2 < jlim)
            def _():
                iwait(jc + 2)
                gather(jc + 2, lax.rem(jc + 2, 3)).start()

                @pl.when(jc + 3 < jlim)
                def _():
                    istart(jc + 3)

            @plsc.parallel_loop(0, CHUNK, step=1, unroll=4)
            def scale_edge(e):
                l = jnp.bitwise_and(e, 15)
                ew_vec = ew_v[r, pl.ds(e - l, 16)]
                sc = _lane_splat(ew_vec, l)
                for j in range(ngrp):
                    rows_v[b, e, pl.ds(j * 16, 16)] = (
                        rows_v[b, e, pl.ds(j * 16, 16)] * sc)

            pltpu.sync_copy(rows_v.at[b], acc.at[sd_v.at[r, 1]], add=True)
            return carry

        lax.fori_loop(0, jlim, body, 0)
        plsc.subcore_barrier()

        for k in range(5):
            pltpu.sync_copy(acc.at[pl.ds(r0 + k * CHUNK, CHUNK)], rows_v.at[0])
            pltpu.sync_copy(rows_v.at[0],
                            out_hbm.at[c, pl.ds(r0 + k * CHUNK, CHUNK)])
        pltpu.sync_copy(acc.at[pl.ds(r0 + 5 * CHUNK, 80)],
                        rows_v.at[0, pl.ds(0, 80)])
        pltpu.sync_copy(rows_v.at[0, pl.ds(0, 80)],
                        out_hbm.at[c, pl.ds(r0 + 5 * CHUNK, 80)])

    return agg


_agg128 = _make_agg(128)

_RB = 2000


def _mm_body(x_ref, w_ref, o_ref):
    o_ref[...] = jnp.dot(x_ref[...], w_ref[...],
                         preferred_element_type=jnp.float32)


def _matmul(x, w):
    n, d = x.shape
    do = w.shape[1]
    return pl.pallas_call(
        _mm_body,
        grid=(n // _RB,),
        in_specs=[
            pl.BlockSpec((_RB, d), lambda i: (i, 0)),
            pl.BlockSpec((d, do), lambda i: (0, 0)),
        ],
        out_specs=pl.BlockSpec((_RB, do), lambda i: (i, 0)),
        out_shape=jax.ShapeDtypeStruct((n, do), jnp.float32),
    )(x, w)


def _fused_body(p_ref, b_ref, w_ref, o_ref):
    h = jax.nn.relu(p_ref[0] + p_ref[1] + b_ref[...])
    o_ref[...] = jnp.dot(h, w_ref[...], preferred_element_type=jnp.float32)


def _fused(p, b, w):
    d = p.shape[2]
    do = w.shape[1]
    return pl.pallas_call(
        _fused_body,
        grid=(N // _RB,),
        in_specs=[
            pl.BlockSpec((2, _RB, d), lambda i: (0, i, 0)),
            pl.BlockSpec((1, d), lambda i: (0, 0)),
            pl.BlockSpec((d, do), lambda i: (0, 0)),
        ],
        out_specs=pl.BlockSpec((_RB, do), lambda i: (i, 0)),
        out_shape=jax.ShapeDtypeStruct((N, do), jnp.float32),
    )(p, b.reshape(1, d), w)


def _act_body(p_ref, b_ref, o_ref):
    o_ref[...] = jax.nn.relu(p_ref[0] + p_ref[1] + b_ref[...])


def _act(p, b):
    d = p.shape[2]
    return pl.pallas_call(
        _act_body,
        grid=(N // _RB,),
        in_specs=[
            pl.BlockSpec((2, _RB, d), lambda i: (0, i, 0)),
            pl.BlockSpec((1, d), lambda i: (0, 0)),
        ],
        out_specs=pl.BlockSpec((_RB, d), lambda i: (i, 0)),
        out_shape=jax.ShapeDtypeStruct((N, d), jnp.float32),
    )(p, b.reshape(1, d))


def _mm_final_body(p_ref, w_ref, b_ref, o_ref):
    o_ref[...] = jnp.dot(p_ref[0] + p_ref[1], w_ref[...],
                         preferred_element_type=jnp.float32) + b_ref[...]


def _mm_final(p, w, b):
    d = p.shape[2]
    do = w.shape[1]
    return pl.pallas_call(
        _mm_final_body,
        grid=(N // _RB,),
        in_specs=[
            pl.BlockSpec((2, _RB, d), lambda i: (0, i, 0)),
            pl.BlockSpec((d, do), lambda i: (0, 0)),
            pl.BlockSpec((1, do), lambda i: (0, 0)),
        ],
        out_specs=pl.BlockSpec((_RB, do), lambda i: (i, 0)),
        out_shape=jax.ShapeDtypeStruct((N, do), jnp.float32),
    )(p, w, b.reshape(1, do))


def kernel(features, edge_index, edge_weight, W0, b0, W1, b1, W2, b2):
    src = edge_index[0].astype(jnp.int32)
    dst = edge_index[1].astype(jnp.int32)
    pad = E_PAD - E
    src = jnp.concatenate([src, jnp.zeros((pad,), jnp.int32)])
    dst = jnp.concatenate([dst, jnp.zeros((pad,), jnp.int32)])
    ew = jnp.concatenate([edge_weight.astype(jnp.float32),
                          jnp.zeros((pad,), jnp.float32)])

    def split(x):
        e0 = 16 * NC0 * CHUNK
        x0 = x[:e0].reshape(16, NC0, CHUNK)
        x0 = jnp.pad(x0, ((0, 0), (0, NCHUNK - NC0), (0, 0)))
        x1 = x[e0:].reshape(16, NC1, CHUNK)
        return jnp.concatenate([x0, x1], axis=0)

    sd = jnp.stack([split(src), split(dst)], axis=2)
    ew = split(ew)

    m0 = _matmul(features, W0)
    p0 = _agg128(m0, sd, ew)
    m1 = _fused(p0, b0, W1)
    p1 = _agg128(m1, sd, ew)
    h2 = _act(p1, b1)
    p2 = _agg128(h2, sd, ew)
    return _mm_final(p2, W2, b2)
